# baseline (speedup 1.0000x reference)
import functools

import jax
import jax.numpy as jnp
from jax import lax
from jax.experimental import pallas as pl
from jax.experimental.pallas import tpu as pltpu
from jax.experimental.pallas import tpu_sc as plsc

_D = 64
_BATCH = 16384
_FIELDS = 26
_NC, _NS = 2, 16
_NW = _NC * _NS
_SPW = _BATCH // _NW
_G = 16
_NGROUP = _SPW // _G

_mesh = plsc.VectorSubcoreMesh(
    core_axis_name="c", subcore_axis_name="s",
    num_cores=_NC, num_subcores=_NS,
)


@functools.partial(
    pl.kernel,
    out_type=jax.ShapeDtypeStruct((_BATCH, _FIELDS, _D), jnp.float32),
    mesh=_mesh,
    scratch_types=[
        pltpu.VMEM((_SPW, _FIELDS), jnp.int32),
        pltpu.VMEM((_G, _FIELDS, _D), jnp.float32),
        pltpu.VMEM((_G, _FIELDS, _D), jnp.float32),
        pltpu.SemaphoreType.DMA,
    ],
    compiler_params=pltpu.CompilerParams(use_tc_tiling_on_sc=False),
)
def _emb_lookup(x_hbm, table_hbm, out_hbm, idx_v, buf_a, buf_b, gsem):
    wid = lax.axis_index("s") * _NC + lax.axis_index("c")
    sbase = wid * _SPW
    pltpu.sync_copy(x_hbm.at[pl.ds(sbase, _SPW)], idx_v)

    def fire(g, buf):
        for i in range(_G):
            pltpu.async_copy(
                table_hbm.at[idx_v.at[g * _G + i]],
                buf.at[i],
                gsem,
            )

    def drain(buf):
        pltpu.make_async_copy(out_hbm.at[pl.ds(0, _G)], buf, gsem).wait()

    fire(0, buf_a)

    @pl.loop(0, _NGROUP, step=2)
    def _group_loop(g):
        fire(g + 1, buf_b)
        drain(buf_a)
        pltpu.sync_copy(buf_a, out_hbm.at[pl.ds(sbase + g * _G, _G)])

        @pl.when(g + 2 < _NGROUP)
        def _():
            fire(g + 2, buf_a)

        drain(buf_b)
        pltpu.sync_copy(buf_b, out_hbm.at[pl.ds(sbase + (g + 1) * _G, _G)])


def kernel(x, table):
    return _emb_lookup(x.astype(jnp.int32), table)


# speedup vs baseline: 1.0273x; 1.0273x over previous
import functools

import jax
import jax.numpy as jnp
from jax import lax
from jax.experimental import pallas as pl
from jax.experimental.pallas import tpu as pltpu
from jax.experimental.pallas import tpu_sc as plsc

_D = 64
_BATCH = 16384
_FIELDS = 26
_NC, _NS = 2, 16
_NW = _NC * _NS
_SPW = _BATCH // _NW
_G = 16
_NGROUP = _SPW // _G

_mesh = plsc.VectorSubcoreMesh(
    core_axis_name="c", subcore_axis_name="s",
    num_cores=_NC, num_subcores=_NS,
)


@functools.partial(
    pl.kernel,
    out_type=jax.ShapeDtypeStruct((_BATCH, _FIELDS, _D), jnp.float32),
    mesh=_mesh,
    scratch_types=[
        pltpu.VMEM((_SPW, _FIELDS), jnp.int32),
        pltpu.VMEM((_G, _FIELDS, 2 * _D), jnp.float32),
        pltpu.VMEM((_G, _FIELDS, 2 * _D), jnp.float32),
        pltpu.SemaphoreType.DMA,
    ],
    compiler_params=pltpu.CompilerParams(use_tc_tiling_on_sc=False),
)
def _emb_lookup(x_hbm, table_hbm, out_hbm, idx_v, buf_a, buf_b, gsem):
    wid = lax.axis_index("s") * _NC + lax.axis_index("c")
    sbase = wid * _SPW
    pltpu.sync_copy(x_hbm.at[pl.ds(sbase, _SPW)], idx_v)

    def fire(g, buf):
        for i in range(_G):
            pltpu.async_copy(
                table_hbm.at[idx_v.at[g * _G + i]],
                buf.at[i],
                gsem,
            )

    def drain(buf):
        pltpu.make_async_copy(table_hbm.at[pl.ds(0, _G * _FIELDS)], buf, gsem).wait()

    def writeback(g, buf):
        pltpu.sync_copy(buf.at[:, :, pl.ds(0, _D)],
                        out_hbm.at[pl.ds(sbase + g * _G, _G)])

    fire(0, buf_a)

    @pl.loop(0, _NGROUP, step=2)
    def _group_loop(g):
        fire(g + 1, buf_b)
        drain(buf_a)
        writeback(g, buf_a)

        @pl.when(g + 2 < _NGROUP)
        def _():
            fire(g + 2, buf_a)

        drain(buf_b)
        writeback(g + 1, buf_b)


def kernel(x, table):
    table_p = jnp.pad(table, ((0, 0), (0, _D)))
    return _emb_lookup(x.astype(jnp.int32), table_p)
